# Initial kernel scaffold; baseline (speedup 1.0000x reference)
import jax
import jax.numpy as jnp
from jax.experimental import pallas as pl


def kernel(x_cat, tables):
    raise NotImplementedError("write your pallas kernel here")


# speedup vs baseline: 3.8337x; 3.8337x over previous
import functools

import jax
import jax.numpy as jnp
from jax import lax
from jax.experimental import pallas as pl
from jax.experimental.pallas import tpu as pltpu
from jax.experimental.pallas import tpu_sc as plsc

NUM_FIELDS = 26
VOCAB = 100
EMB_DIM = 64
BATCH = 16384

NC = 2
NS = 16
NW = NC * NS
LANES = 16

ROWS = BATCH * NUM_FIELDS
RPW = ROWS // NW
CHUNK = 128
NCH = RPW // CHUNK

_mesh = plsc.VectorSubcoreMesh(
    core_axis_name="c", subcore_axis_name="s", num_cores=NC, num_subcores=NS
)


@functools.partial(
    pl.kernel,
    out_type=jax.ShapeDtypeStruct((ROWS, EMB_DIM), jnp.float32),
    mesh=_mesh,
    scratch_types=[
        pltpu.VMEM((RPW,), jnp.int32),
        pltpu.VMEM((NCH, CHUNK), jnp.int32),
        pltpu.VMEM((CHUNK, EMB_DIM), jnp.float32),
        pltpu.SemaphoreType.DMA,
    ],
    compiler_params=pltpu.CompilerParams(use_tc_tiling_on_sc=False),
)
def _emb_lookup(x_hbm, tbl_hbm, out_hbm, raw_v, idx_v, buf_v, sem):
    wid = lax.axis_index("s") * NC + lax.axis_index("c")
    base = wid * RPW

    pltpu.sync_copy(x_hbm.at[pl.ds(base, RPW)], raw_v)

    def compute(c, carry):
        for k in range(CHUNK // LANES):
            j0 = c * CHUNK + k * LANES
            j = j0 + lax.iota(jnp.int32, LANES)
            fld = lax.rem(j, NUM_FIELDS)
            idx_v[c, pl.ds(k * LANES, LANES)] = (
                raw_v[pl.ds(j0, LANES)] + fld * VOCAB
            )
        return carry

    lax.fori_loop(0, NCH, compute, 0)

    def gather(c, carry):
        pltpu.async_copy(tbl_hbm.at[idx_v.at[c]], buf_v, sem).wait()
        pltpu.sync_copy(buf_v, out_hbm.at[pl.ds(base + c * CHUNK, CHUNK)])
        return carry

    lax.fori_loop(0, NCH, gather, 0)


def kernel(x_cat, tables):
    x_flat = x_cat.reshape(-1).astype(jnp.int32)
    tbl = tables.reshape(NUM_FIELDS * VOCAB, EMB_DIM)
    out = _emb_lookup(x_flat, tbl)
    return out.reshape(BATCH, NUM_FIELDS, EMB_DIM)


# speedup vs baseline: 4.3935x; 1.1460x over previous
import functools

import jax
import jax.numpy as jnp
from jax import lax
from jax.experimental import pallas as pl
from jax.experimental.pallas import tpu as pltpu
from jax.experimental.pallas import tpu_sc as plsc

NUM_FIELDS = 26
VOCAB = 100
EMB_DIM = 64
BATCH = 16384

NC = 2
NS = 16
NW = NC * NS
LANES = 16

ROWS = BATCH * NUM_FIELDS
RPW = ROWS // NW
CHUNK = 128
NCH = RPW // CHUNK
SUB = 4
BIG = CHUNK * SUB
NBIG = RPW // BIG

_mesh = plsc.VectorSubcoreMesh(
    core_axis_name="c", subcore_axis_name="s", num_cores=NC, num_subcores=NS
)


@functools.partial(
    pl.kernel,
    out_type=jax.ShapeDtypeStruct((ROWS, EMB_DIM), jnp.float32),
    mesh=_mesh,
    scratch_types=[
        pltpu.VMEM((RPW,), jnp.int32),
        pltpu.VMEM((NCH, CHUNK), jnp.int32),
        pltpu.VMEM((BIG, EMB_DIM), jnp.float32),
        pltpu.VMEM((BIG, EMB_DIM), jnp.float32),
        pltpu.SemaphoreType.DMA,
        pltpu.SemaphoreType.DMA,
    ],
    compiler_params=pltpu.CompilerParams(use_tc_tiling_on_sc=False),
)
def _emb_lookup(x_hbm, tbl_hbm, out_hbm, raw_v, idx_v, buf0_v, buf1_v, sem0, sem1):
    wid = lax.axis_index("s") * NC + lax.axis_index("c")
    base = wid * RPW

    pltpu.sync_copy(x_hbm.at[pl.ds(base, RPW)], raw_v)

    def compute(c, fld):
        for k in range(CHUNK // LANES):
            j0 = c * CHUNK + k * LANES
            idx_v[c, pl.ds(k * LANES, LANES)] = (
                raw_v[pl.ds(j0, LANES)] + fld * VOCAB
            )
            t = fld + LANES
            fld = lax.select(t >= NUM_FIELDS, t - NUM_FIELDS, t)
        return fld

    lax.fori_loop(0, NCH, compute, lax.iota(jnp.int32, LANES))

    bufs = (buf0_v, buf1_v)
    sems = (sem0, sem1)

    def fire(g, b):
        cps = []
        for s in range(SUB):
            c = g * SUB + s
            cps.append(
                pltpu.async_copy(
                    tbl_hbm.at[idx_v.at[c]],
                    bufs[b].at[pl.ds(s * CHUNK, CHUNK)],
                    sems[b],
                )
            )
        return cps

    pending = fire(0, 0)
    for g in range(NBIG):
        b = g % 2
        nxt = fire(g + 1, 1 - b) if g + 1 < NBIG else []
        for cp in pending:
            cp.wait()
        pending = nxt
        pltpu.sync_copy(bufs[b], out_hbm.at[pl.ds(base + g * BIG, BIG)])


def kernel(x_cat, tables):
    x_flat = x_cat.reshape(-1).astype(jnp.int32)
    tbl = tables.reshape(NUM_FIELDS * VOCAB, EMB_DIM)
    out = _emb_lookup(x_flat, tbl)
    return out.reshape(BATCH, NUM_FIELDS, EMB_DIM)
